# baseline (speedup 1.0000x reference)
import functools

import jax
import jax.numpy as jnp
from jax import lax
from jax.experimental import pallas as pl
from jax.experimental.pallas import tpu as pltpu
from jax.experimental.pallas import tpu_sc as plsc

K = 32
QB = 256
CB = 2048
CH = 128
CPT = CB // CH
NEG = float("-inf")
BIG_I = 2**30
NW = 32
NC = 2


def _a_kernel(q_ref, xt_ref, d_ref, g2_ref, *, n_total):
    j = pl.program_id(0)
    q = q_ref[...]
    xt = xt_ref[...]
    xx = jnp.sum(q * q, axis=1, keepdims=True)
    yy = jnp.sum(xt * xt, axis=1)
    mm = lax.dot_general(q, xt, (((1,), (1,)), ((), ())),
                         preferred_element_type=jnp.float32)
    dist = jnp.sqrt(jnp.maximum(xx + yy[None, :] - 2.0 * mm, 0.0))
    lane = lax.broadcasted_iota(jnp.int32, (QB, CB), 1)
    dist = jnp.where(j * CB + lane < n_total, dist, NEG)
    d3 = dist.reshape(QB, CPT, CH)
    d_ref[...] = d3
    g2_ref[0] = jnp.max(d3, axis=2)


def _a2_kernel(g2_ref, cvals_ref, cids_ref, g2_s, *, nch):
    g2_s[...] = g2_ref[...]
    lane = lax.broadcasted_iota(jnp.int32, (QB, nch), 1)
    lane_k = lax.broadcasted_iota(jnp.int32, (QB, K), 1)

    def body(t, carry):
        cv, ci = carry
        g = g2_s[...]
        m = jnp.max(g, axis=1)
        ism = g == m[:, None]
        loc = jnp.min(jnp.where(ism, lane, BIG_I), axis=1)
        g2_s[...] = jnp.where(lane == loc[:, None], NEG, g)
        onk = lane_k == t
        cv = jnp.where(onk, m[:, None], cv)
        ci = jnp.where(onk, loc[:, None], ci)
        return cv, ci

    cv, ci = lax.fori_loop(
        0, K, body,
        (jnp.full((QB, K), NEG, jnp.float32), jnp.zeros((QB, K), jnp.int32)))
    cvals_ref[...] = cv
    cids_ref[...] = ci


def _sc_body(dv_hbm, cids_hbm, gath_hbm, cid_v, gi_v, rows_v, sem,
             *, nch, nqw):
    wid = lax.axis_index("s") * NC + lax.axis_index("c")

    def qbody(qi, _):
        q = wid * nqw + qi
        pltpu.sync_copy(cids_hbm.at[q], cid_v)
        c0 = cid_v[pl.ds(0, 16)]
        c1 = cid_v[pl.ds(16, 16)]
        gi_v[pl.ds(0, 16)] = c0 + q * nch
        gi_v[pl.ds(16, 16)] = c1 + q * nch
        pltpu.async_copy(dv_hbm.at[gi_v], rows_v, sem).wait()
        pltpu.sync_copy(rows_v, gath_hbm.at[q])
        return 0

    lax.fori_loop(0, nqw, qbody, 0)


def _c_kernel(g_ref, cids_ref, vals_ref, idx_ref, *, ncand):
    lane_k = lax.broadcasted_iota(jnp.int32, (QB, K), 1)
    sub = lax.broadcasted_iota(jnp.int32, (QB, K, CH), 2)
    ci = cids_ref[...]
    kidx = (ci[:, :, None] * CH + sub).reshape(QB, ncand)
    v = g_ref[...].reshape(QB, ncand)

    def body(t, carry):
        v, ov, oi = carry
        m = jnp.max(v, axis=1)
        ism = v == m[:, None]
        li = jnp.min(jnp.where(ism, kidx, BIG_I), axis=1)
        sel = ism & (kidx == li[:, None])
        v = jnp.where(sel, NEG, v)
        onk = lane_k == t
        ov = jnp.where(onk, m[:, None], ov)
        oi = jnp.where(onk, li[:, None], oi)
        return v, ov, oi

    _, ov, oi = lax.fori_loop(
        0, K, body,
        (v, jnp.full((QB, K), NEG, jnp.float32), jnp.zeros((QB, K), jnp.int32)))
    vals_ref[...] = ov
    idx_ref[...] = oi


def kernel(x_test, x_train, y_train):
    del y_train
    q, d = x_test.shape
    n, _ = x_train.shape
    nkt = -(-n // CB)
    npad = nkt * CB
    if npad != n:
        x_train = jnp.pad(x_train, ((0, npad - n), (0, 0)))
    nqt = q // QB
    nch = nkt * CPT
    nqw = q // NW

    dmat, g2 = pl.pallas_call(
        functools.partial(_a_kernel, n_total=n),
        grid=(nkt, nqt),
        in_specs=[
            pl.BlockSpec((QB, d), lambda j, i: (i, 0)),
            pl.BlockSpec((CB, d), lambda j, i: (j, 0)),
        ],
        out_specs=[
            pl.BlockSpec((QB, CPT, CH), lambda j, i: (i, j, 0)),
            pl.BlockSpec((1, QB, CPT), lambda j, i: (j, i, 0)),
        ],
        out_shape=[
            jax.ShapeDtypeStruct((q, nch, CH), jnp.float32),
            jax.ShapeDtypeStruct((nkt, q, CPT), jnp.float32),
        ],
        compiler_params=pltpu.CompilerParams(
            dimension_semantics=("arbitrary", "parallel")),
    )(x_test, x_train)

    g2t = jnp.transpose(g2, (1, 0, 2)).reshape(q, nch)

    cvals, cids = pl.pallas_call(
        functools.partial(_a2_kernel, nch=nch),
        grid=(nqt,),
        in_specs=[pl.BlockSpec((QB, nch), lambda i: (i, 0))],
        out_specs=[
            pl.BlockSpec((QB, K), lambda i: (i, 0)),
            pl.BlockSpec((QB, K), lambda i: (i, 0)),
        ],
        out_shape=[
            jax.ShapeDtypeStruct((q, K), jnp.float32),
            jax.ShapeDtypeStruct((q, K), jnp.int32),
        ],
        scratch_shapes=[pltpu.VMEM((QB, nch), jnp.float32)],
        compiler_params=pltpu.CompilerParams(
            dimension_semantics=("parallel",)),
    )(g2t)

    dv = dmat.reshape(q * nch, CH)
    ncand = K * CH

    sc_fn = pl.kernel(
        functools.partial(_sc_body, nch=nch, nqw=nqw),
        out_type=[jax.ShapeDtypeStruct((q, K, CH), jnp.float32)],
        mesh=plsc.VectorSubcoreMesh(core_axis_name="c", subcore_axis_name="s"),
        scratch_types=[
            pltpu.VMEM((K,), jnp.int32),
            pltpu.VMEM((K,), jnp.int32),
            pltpu.VMEM((K, CH), jnp.float32),
            pltpu.SemaphoreType.DMA,
        ],
    )
    (gath,) = sc_fn(dv, cids)

    vals, idx = pl.pallas_call(
        functools.partial(_c_kernel, ncand=ncand),
        grid=(nqt,),
        in_specs=[
            pl.BlockSpec((QB, K, CH), lambda i: (i, 0, 0)),
            pl.BlockSpec((QB, K), lambda i: (i, 0)),
        ],
        out_specs=[
            pl.BlockSpec((QB, K), lambda i: (i, 0)),
            pl.BlockSpec((QB, K), lambda i: (i, 0)),
        ],
        out_shape=[
            jax.ShapeDtypeStruct((q, K), jnp.float32),
            jax.ShapeDtypeStruct((q, K), jnp.int32),
        ],
        compiler_params=pltpu.CompilerParams(
            dimension_semantics=("parallel",)),
    )(gath, cids)
    return vals, idx


# speedup vs baseline: 1.0738x; 1.0738x over previous
import functools

import jax
import jax.numpy as jnp
from jax import lax
from jax.experimental import pallas as pl
from jax.experimental.pallas import tpu as pltpu
from jax.experimental.pallas import tpu_sc as plsc

K = 32
QB = 256
CB = 2048
CH = 128
CPT = CB // CH
NEG = float("-inf")
BIG_I = 2**30
NW = 32
NC = 2


def _a_kernel(q_ref, xt_ref, d_ref, g2_ref, *, n_total):
    j = pl.program_id(0)
    q = q_ref[...]
    xt = xt_ref[...]
    xx = jnp.sum(q * q, axis=1, keepdims=True)
    yy = jnp.sum(xt * xt, axis=1)
    mm = lax.dot_general(q, xt, (((1,), (1,)), ((), ())),
                         preferred_element_type=jnp.float32)
    d2 = jnp.maximum(xx + yy[None, :] - 2.0 * mm, 0.0)
    lane = lax.broadcasted_iota(jnp.int32, (QB, CB), 1)
    d2 = jnp.where(j * CB + lane < n_total, d2, -1.0)
    d3 = d2.reshape(QB, CPT, CH)
    d_ref[...] = d3
    g2_ref[0] = jnp.max(d3, axis=2)


def _a2_kernel(g2_ref, cvals_ref, cids_ref, g2_s, *, nch):
    graw = g2_ref[...]
    g2_s[...] = jnp.where(graw >= 0.0, jnp.sqrt(graw), NEG)
    lane = lax.broadcasted_iota(jnp.int32, (QB, nch), 1)
    lane_k = lax.broadcasted_iota(jnp.int32, (QB, K), 1)

    def body(t, carry):
        cv, ci = carry
        g = g2_s[...]
        m = jnp.max(g, axis=1)
        ism = g == m[:, None]
        loc = jnp.min(jnp.where(ism, lane, BIG_I), axis=1)
        g2_s[...] = jnp.where(lane == loc[:, None], NEG, g)
        onk = lane_k == t
        cv = jnp.where(onk, m[:, None], cv)
        ci = jnp.where(onk, loc[:, None], ci)
        return cv, ci

    cv, ci = lax.fori_loop(
        0, K, body,
        (jnp.full((QB, K), NEG, jnp.float32), jnp.zeros((QB, K), jnp.int32)))
    cvals_ref[...] = cv
    cids_ref[...] = ci


def _sc_body(dv_hbm, cids_hbm, gath_hbm, cid_v, gi_v, rows_v, sem,
             *, nch, nqw):
    wid = lax.axis_index("s") * NC + lax.axis_index("c")

    def qbody(qi, _):
        q = wid * nqw + qi
        pltpu.sync_copy(cids_hbm.at[q], cid_v)
        c0 = cid_v[pl.ds(0, 16)]
        c1 = cid_v[pl.ds(16, 16)]
        gi_v[pl.ds(0, 16)] = c0 + q * nch
        gi_v[pl.ds(16, 16)] = c1 + q * nch
        pltpu.async_copy(dv_hbm.at[gi_v], rows_v, sem).wait()
        pltpu.sync_copy(rows_v, gath_hbm.at[q])
        return 0

    lax.fori_loop(0, nqw, qbody, 0)


def _c_kernel(g_ref, cids_ref, vals_ref, idx_ref, *, ncand):
    lane_k = lax.broadcasted_iota(jnp.int32, (QB, K), 1)
    sub = lax.broadcasted_iota(jnp.int32, (QB, K, CH), 2)
    ci = cids_ref[...]
    kidx = (ci[:, :, None] * CH + sub).reshape(QB, ncand)
    g = g_ref[...].reshape(QB, ncand)
    v = jnp.where(g >= 0.0, jnp.sqrt(g), NEG)

    def body(t, carry):
        v, ov, oi = carry
        m = jnp.max(v, axis=1)
        ism = v == m[:, None]
        li = jnp.min(jnp.where(ism, kidx, BIG_I), axis=1)
        sel = ism & (kidx == li[:, None])
        v = jnp.where(sel, NEG, v)
        onk = lane_k == t
        ov = jnp.where(onk, m[:, None], ov)
        oi = jnp.where(onk, li[:, None], oi)
        return v, ov, oi

    _, ov, oi = lax.fori_loop(
        0, K, body,
        (v, jnp.full((QB, K), NEG, jnp.float32), jnp.zeros((QB, K), jnp.int32)))
    vals_ref[...] = ov
    idx_ref[...] = oi


def kernel(x_test, x_train, y_train):
    del y_train
    q, d = x_test.shape
    n, _ = x_train.shape
    nkt = -(-n // CB)
    npad = nkt * CB
    if npad != n:
        x_train = jnp.pad(x_train, ((0, npad - n), (0, 0)))
    nqt = q // QB
    nch = nkt * CPT
    nqw = q // NW

    dmat, g2 = pl.pallas_call(
        functools.partial(_a_kernel, n_total=n),
        grid=(nkt, nqt),
        in_specs=[
            pl.BlockSpec((QB, d), lambda j, i: (i, 0)),
            pl.BlockSpec((CB, d), lambda j, i: (j, 0)),
        ],
        out_specs=[
            pl.BlockSpec((QB, CPT, CH), lambda j, i: (i, j, 0)),
            pl.BlockSpec((1, QB, CPT), lambda j, i: (j, i, 0)),
        ],
        out_shape=[
            jax.ShapeDtypeStruct((q, nch, CH), jnp.float32),
            jax.ShapeDtypeStruct((nkt, q, CPT), jnp.float32),
        ],
        compiler_params=pltpu.CompilerParams(
            dimension_semantics=("arbitrary", "parallel")),
    )(x_test, x_train)

    g2t = jnp.transpose(g2, (1, 0, 2)).reshape(q, nch)

    cvals, cids = pl.pallas_call(
        functools.partial(_a2_kernel, nch=nch),
        grid=(nqt,),
        in_specs=[pl.BlockSpec((QB, nch), lambda i: (i, 0))],
        out_specs=[
            pl.BlockSpec((QB, K), lambda i: (i, 0)),
            pl.BlockSpec((QB, K), lambda i: (i, 0)),
        ],
        out_shape=[
            jax.ShapeDtypeStruct((q, K), jnp.float32),
            jax.ShapeDtypeStruct((q, K), jnp.int32),
        ],
        scratch_shapes=[pltpu.VMEM((QB, nch), jnp.float32)],
        compiler_params=pltpu.CompilerParams(
            dimension_semantics=("parallel",)),
    )(g2t)

    dv = dmat.reshape(q * nch, CH)
    ncand = K * CH

    sc_fn = pl.kernel(
        functools.partial(_sc_body, nch=nch, nqw=nqw),
        out_type=[jax.ShapeDtypeStruct((q, K, CH), jnp.float32)],
        mesh=plsc.VectorSubcoreMesh(core_axis_name="c", subcore_axis_name="s"),
        scratch_types=[
            pltpu.VMEM((K,), jnp.int32),
            pltpu.VMEM((K,), jnp.int32),
            pltpu.VMEM((K, CH), jnp.float32),
            pltpu.SemaphoreType.DMA,
        ],
    )
    (gath,) = sc_fn(dv, cids)

    vals, idx = pl.pallas_call(
        functools.partial(_c_kernel, ncand=ncand),
        grid=(nqt,),
        in_specs=[
            pl.BlockSpec((QB, K, CH), lambda i: (i, 0, 0)),
            pl.BlockSpec((QB, K), lambda i: (i, 0)),
        ],
        out_specs=[
            pl.BlockSpec((QB, K), lambda i: (i, 0)),
            pl.BlockSpec((QB, K), lambda i: (i, 0)),
        ],
        out_shape=[
            jax.ShapeDtypeStruct((q, K), jnp.float32),
            jax.ShapeDtypeStruct((q, K), jnp.int32),
        ],
        compiler_params=pltpu.CompilerParams(
            dimension_semantics=("parallel",)),
    )(gath, cids)
    return vals, idx


# speedup vs baseline: 1.1335x; 1.0556x over previous
import functools

import jax
import jax.numpy as jnp
from jax import lax
from jax.experimental import pallas as pl
from jax.experimental.pallas import tpu as pltpu
from jax.experimental.pallas import tpu_sc as plsc

K = 32
QB = 256
CB = 2048
CH = 128
CPT = CB // CH
NEG = float("-inf")
BIG_I = 2**30
NW = 32
NC = 2


def _a_kernel(q_ref, xt_ref, d_ref, g2_ref, *, n_total):
    j = pl.program_id(0)
    q = q_ref[...]
    xt = xt_ref[...]
    xx = jnp.sum(q * q, axis=1, keepdims=True)
    yy = jnp.sum(xt * xt, axis=1)
    mm = lax.dot_general(q, xt, (((1,), (1,)), ((), ())),
                         preferred_element_type=jnp.float32)
    d2 = jnp.maximum(xx + yy[None, :] - 2.0 * mm, 0.0)
    lane = lax.broadcasted_iota(jnp.int32, (QB, CB), 1)
    d2 = jnp.where(j * CB + lane < n_total, d2, -1.0)
    d3 = d2.reshape(QB, CPT, CH)
    d_ref[...] = d3
    g2_ref[0] = jnp.max(d3, axis=2)


def _a2_kernel(g2_ref, cvals_ref, cids_ref, g2_s, *, nch):
    graw = g2_ref[...]
    g2_s[...] = jnp.where(graw >= 0.0, jnp.sqrt(graw), NEG)
    lane = lax.broadcasted_iota(jnp.int32, (QB, nch), 1)
    lane_k = lax.broadcasted_iota(jnp.int32, (QB, K), 1)

    def body(t, carry):
        cv, ci = carry
        g = g2_s[...]
        m = jnp.max(g, axis=1)
        ism = g == m[:, None]
        loc = jnp.min(jnp.where(ism, lane, BIG_I), axis=1)
        g2_s[...] = jnp.where(lane == loc[:, None], NEG, g)
        onk = lane_k == t
        cv = jnp.where(onk, m[:, None], cv)
        ci = jnp.where(onk, loc[:, None], ci)
        return cv, ci

    cv, ci = lax.fori_loop(
        0, K, body,
        (jnp.full((QB, K), NEG, jnp.float32), jnp.zeros((QB, K), jnp.int32)))
    cvals_ref[...] = cv
    cids_ref[...] = ci


def _sc_body(dv_hbm, cids_hbm, gath_hbm, cid_v, gi_v, rows_v, sem,
             *, nch, nqw, qbase):
    wid = lax.axis_index("s") * NC + lax.axis_index("c")

    def qbody(qi, _):
        q = wid * nqw + qi
        pltpu.sync_copy(cids_hbm.at[q], cid_v)
        c0 = cid_v[pl.ds(0, 16)]
        c1 = cid_v[pl.ds(16, 16)]
        gi_v[pl.ds(0, 16)] = c0 + (q + qbase) * nch
        gi_v[pl.ds(16, 16)] = c1 + (q + qbase) * nch
        pltpu.async_copy(dv_hbm.at[gi_v], rows_v, sem).wait()
        pltpu.sync_copy(rows_v, gath_hbm.at[q])
        return 0

    lax.fori_loop(0, nqw, qbody, 0)


def _c_kernel(g_ref, cids_ref, vals_ref, idx_ref, *, ncand):
    lane_k = lax.broadcasted_iota(jnp.int32, (QB, K), 1)
    sub = lax.broadcasted_iota(jnp.int32, (QB, K, CH), 2)
    ci = cids_ref[...]
    kidx = (ci[:, :, None] * CH + sub).reshape(QB, ncand)
    g = g_ref[...].reshape(QB, ncand)
    v = jnp.where(g >= 0.0, jnp.sqrt(g), NEG)

    def body(t, carry):
        v, ov, oi = carry
        m = jnp.max(v, axis=1)
        ism = v == m[:, None]
        li = jnp.min(jnp.where(ism, kidx, BIG_I), axis=1)
        sel = ism & (kidx == li[:, None])
        v = jnp.where(sel, NEG, v)
        onk = lane_k == t
        ov = jnp.where(onk, m[:, None], ov)
        oi = jnp.where(onk, li[:, None], oi)
        return v, ov, oi

    _, ov, oi = lax.fori_loop(
        0, K, body,
        (v, jnp.full((QB, K), NEG, jnp.float32), jnp.zeros((QB, K), jnp.int32)))
    vals_ref[...] = ov
    idx_ref[...] = oi


def kernel(x_test, x_train, y_train):
    del y_train
    q, d = x_test.shape
    n, _ = x_train.shape
    nkt = -(-n // CB)
    npad = nkt * CB
    if npad != n:
        x_train = jnp.pad(x_train, ((0, npad - n), (0, 0)))
    nqt = q // QB
    nch = nkt * CPT
    nqw = q // NW

    dmat, g2 = pl.pallas_call(
        functools.partial(_a_kernel, n_total=n),
        grid=(nkt, nqt),
        in_specs=[
            pl.BlockSpec((QB, d), lambda j, i: (i, 0)),
            pl.BlockSpec((CB, d), lambda j, i: (j, 0)),
        ],
        out_specs=[
            pl.BlockSpec((QB, CPT, CH), lambda j, i: (i, j, 0)),
            pl.BlockSpec((1, QB, CPT), lambda j, i: (j, i, 0)),
        ],
        out_shape=[
            jax.ShapeDtypeStruct((q, nch, CH), jnp.float32),
            jax.ShapeDtypeStruct((nkt, q, CPT), jnp.float32),
        ],
        compiler_params=pltpu.CompilerParams(
            dimension_semantics=("arbitrary", "parallel")),
    )(x_test, x_train)

    g2t = jnp.transpose(g2, (1, 0, 2)).reshape(q, nch)
    dv = dmat.reshape(q * nch, CH)
    ncand = K * CH

    halves = []
    hq = q // 2
    for h in range(2):
        g2h = jax.lax.slice_in_dim(g2t, h * hq, (h + 1) * hq, axis=0)
        cvals, cids = pl.pallas_call(
            functools.partial(_a2_kernel, nch=nch),
            grid=(hq // QB,),
            in_specs=[pl.BlockSpec((QB, nch), lambda i: (i, 0))],
            out_specs=[
                pl.BlockSpec((QB, K), lambda i: (i, 0)),
                pl.BlockSpec((QB, K), lambda i: (i, 0)),
            ],
            out_shape=[
                jax.ShapeDtypeStruct((hq, K), jnp.float32),
                jax.ShapeDtypeStruct((hq, K), jnp.int32),
            ],
            scratch_shapes=[pltpu.VMEM((QB, nch), jnp.float32)],
            compiler_params=pltpu.CompilerParams(
                dimension_semantics=("parallel",)),
        )(g2h)

        sc_fn = pl.kernel(
            functools.partial(_sc_body, nch=nch, nqw=hq // NW, qbase=h * hq),
            out_type=[jax.ShapeDtypeStruct((hq, K, CH), jnp.float32)],
            mesh=plsc.VectorSubcoreMesh(core_axis_name="c",
                                        subcore_axis_name="s"),
            scratch_types=[
                pltpu.VMEM((K,), jnp.int32),
                pltpu.VMEM((K,), jnp.int32),
                pltpu.VMEM((K, CH), jnp.float32),
                pltpu.SemaphoreType.DMA,
            ],
        )
        (gath,) = sc_fn(dv, cids)

        vh, ih = pl.pallas_call(
            functools.partial(_c_kernel, ncand=ncand),
            grid=(hq // QB,),
            in_specs=[
                pl.BlockSpec((QB, K, CH), lambda i: (i, 0, 0)),
                pl.BlockSpec((QB, K), lambda i: (i, 0)),
            ],
            out_specs=[
                pl.BlockSpec((QB, K), lambda i: (i, 0)),
                pl.BlockSpec((QB, K), lambda i: (i, 0)),
            ],
            out_shape=[
                jax.ShapeDtypeStruct((hq, K), jnp.float32),
                jax.ShapeDtypeStruct((hq, K), jnp.int32),
            ],
            compiler_params=pltpu.CompilerParams(
                dimension_semantics=("parallel",)),
        )(gath, cids)
        halves.append((vh, ih))

    vals = jnp.concatenate([halves[0][0], halves[1][0]], axis=0)
    idx = jnp.concatenate([halves[0][1], halves[1][1]], axis=0)
    return vals, idx


# speedup vs baseline: 1.1343x; 1.0007x over previous
import functools

import jax
import jax.numpy as jnp
from jax import lax
from jax.experimental import pallas as pl
from jax.experimental.pallas import tpu as pltpu
from jax.experimental.pallas import tpu_sc as plsc

K = 32
QB = 256
CB = 2048
CH = 128
CPT = CB // CH
NEG = float("-inf")
BIG_I = 2**30
NW = 32
NC = 2


def _a_kernel(q_ref, xt_ref, d_ref, g2_ref, *, n_total):
    j = pl.program_id(0)
    q = q_ref[...]
    xt = xt_ref[...]
    xx = jnp.sum(q * q, axis=1, keepdims=True)
    yy = jnp.sum(xt * xt, axis=1)
    mm = lax.dot_general(q, xt, (((1,), (1,)), ((), ())),
                         preferred_element_type=jnp.float32)
    d2 = jnp.maximum(xx + yy[None, :] - 2.0 * mm, 0.0)
    lane = lax.broadcasted_iota(jnp.int32, (QB, CB), 1)
    d2 = jnp.where(j * CB + lane < n_total, d2, -1.0)
    d3 = d2.reshape(QB, CPT, CH)
    d_ref[...] = d3
    g2_ref[0] = jnp.max(d3, axis=2)


def _a2_kernel(g2_ref, cvals_ref, cids_ref, g2_s, *, nch):
    graw = g2_ref[...]
    g2_s[...] = jnp.where(graw >= 0.0, jnp.sqrt(graw), NEG)
    lane = lax.broadcasted_iota(jnp.int32, (QB, nch), 1)
    lane_k = lax.broadcasted_iota(jnp.int32, (QB, K), 1)

    def body(t, carry):
        cv, ci = carry
        g = g2_s[...]
        m = jnp.max(g, axis=1)
        ism = g == m[:, None]
        loc = jnp.min(jnp.where(ism, lane, BIG_I), axis=1)
        g2_s[...] = jnp.where(lane == loc[:, None], NEG, g)
        onk = lane_k == t
        cv = jnp.where(onk, m[:, None], cv)
        ci = jnp.where(onk, loc[:, None], ci)
        return cv, ci

    cv, ci = lax.fori_loop(
        0, K, body,
        (jnp.full((QB, K), NEG, jnp.float32), jnp.zeros((QB, K), jnp.int32)))
    cvals_ref[...] = cv
    cids_ref[...] = ci


def _sc_body(dv_hbm, cids_hbm, gath_hbm, cid_v, gi_v, rows_v, sem,
             *, nch, nqw, qbase):
    wid = lax.axis_index("s") * NC + lax.axis_index("c")

    def qbody(qi, _):
        q = wid * nqw + qi
        pltpu.sync_copy(cids_hbm.at[q], cid_v)
        c0 = cid_v[pl.ds(0, 16)]
        c1 = cid_v[pl.ds(16, 16)]
        gi_v[pl.ds(0, 16)] = c0 + (q + qbase) * nch
        gi_v[pl.ds(16, 16)] = c1 + (q + qbase) * nch
        pltpu.async_copy(dv_hbm.at[gi_v], rows_v, sem).wait()
        pltpu.sync_copy(rows_v, gath_hbm.at[q])
        return 0

    lax.fori_loop(0, nqw, qbody, 0)


def _c_kernel(g_ref, cids_ref, vals_ref, idx_ref, *, ncand):
    lane_k = lax.broadcasted_iota(jnp.int32, (QB, K), 1)
    sub = lax.broadcasted_iota(jnp.int32, (QB, K, CH), 2)
    ci = cids_ref[...]
    kidx = (ci[:, :, None] * CH + sub).reshape(QB, ncand)
    g = g_ref[...].reshape(QB, ncand)
    v = jnp.where(g >= 0.0, jnp.sqrt(g), NEG)

    def body(t, carry):
        v, ov, oi = carry
        m = jnp.max(v, axis=1)
        ism = v == m[:, None]
        li = jnp.min(jnp.where(ism, kidx, BIG_I), axis=1)
        sel = ism & (kidx == li[:, None])
        v = jnp.where(sel, NEG, v)
        onk = lane_k == t
        ov = jnp.where(onk, m[:, None], ov)
        oi = jnp.where(onk, li[:, None], oi)
        return v, ov, oi

    _, ov, oi = lax.fori_loop(
        0, K, body,
        (v, jnp.full((QB, K), NEG, jnp.float32), jnp.zeros((QB, K), jnp.int32)))
    vals_ref[...] = ov
    idx_ref[...] = oi


def kernel(x_test, x_train, y_train):
    del y_train
    q, d = x_test.shape
    n, _ = x_train.shape
    nkt = -(-n // CB)
    npad = nkt * CB
    if npad != n:
        x_train = jnp.pad(x_train, ((0, npad - n), (0, 0)))
    nqt = q // QB
    nch = nkt * CPT

    dmat, g2 = pl.pallas_call(
        functools.partial(_a_kernel, n_total=n),
        grid=(nkt, nqt),
        in_specs=[
            pl.BlockSpec((QB, d), lambda j, i: (i, 0)),
            pl.BlockSpec((CB, d), lambda j, i: (j, 0)),
        ],
        out_specs=[
            pl.BlockSpec((QB, CPT, CH), lambda j, i: (i, j, 0)),
            pl.BlockSpec((1, QB, CPT), lambda j, i: (j, i, 0)),
        ],
        out_shape=[
            jax.ShapeDtypeStruct((q, nch, CH), jnp.float32),
            jax.ShapeDtypeStruct((nkt, q, CPT), jnp.float32),
        ],
        compiler_params=pltpu.CompilerParams(
            dimension_semantics=("arbitrary", "parallel")),
    )(x_test, x_train)

    g2t = jnp.transpose(g2, (1, 0, 2)).reshape(q, nch)
    dv = dmat.reshape(q * nch, CH)
    ncand = K * CH

    halves = []
    hq = q // 2
    for h in range(2):
        g2h = jax.lax.slice_in_dim(g2t, h * hq, (h + 1) * hq, axis=0)
        cvals, cids = pl.pallas_call(
            functools.partial(_a2_kernel, nch=nch),
            grid=(hq // QB,),
            in_specs=[pl.BlockSpec((QB, nch), lambda i: (i, 0))],
            out_specs=[
                pl.BlockSpec((QB, K), lambda i: (i, 0)),
                pl.BlockSpec((QB, K), lambda i: (i, 0)),
            ],
            out_shape=[
                jax.ShapeDtypeStruct((hq, K), jnp.float32),
                jax.ShapeDtypeStruct((hq, K), jnp.int32),
            ],
            scratch_shapes=[pltpu.VMEM((QB, nch), jnp.float32)],
            compiler_params=pltpu.CompilerParams(
                dimension_semantics=("parallel",)),
        )(g2h)

        sc_fn = pl.kernel(
            functools.partial(_sc_body, nch=nch, nqw=hq // NW, qbase=h * hq),
            out_type=[jax.ShapeDtypeStruct((hq, K, CH), jnp.float32)],
            mesh=plsc.VectorSubcoreMesh(core_axis_name="c",
                                        subcore_axis_name="s"),
            scratch_types=[
                pltpu.VMEM((K,), jnp.int32),
                pltpu.VMEM((K,), jnp.int32),
                pltpu.VMEM((K, CH), jnp.float32),
                pltpu.SemaphoreType.DMA,
            ],
        )
        (gath,) = sc_fn(dv, cids)

        vh, ih = pl.pallas_call(
            functools.partial(_c_kernel, ncand=ncand),
            grid=(hq // QB,),
            in_specs=[
                pl.BlockSpec((QB, K, CH), lambda i: (i, 0, 0)),
                pl.BlockSpec((QB, K), lambda i: (i, 0)),
            ],
            out_specs=[
                pl.BlockSpec((QB, K), lambda i: (i, 0)),
                pl.BlockSpec((QB, K), lambda i: (i, 0)),
            ],
            out_shape=[
                jax.ShapeDtypeStruct((hq, K), jnp.float32),
                jax.ShapeDtypeStruct((hq, K), jnp.int32),
            ],
            compiler_params=pltpu.CompilerParams(
                dimension_semantics=("parallel",)),
        )(gath, cids)
        halves.append((vh, ih))

    vals = jnp.concatenate([halves[0][0], halves[1][0]], axis=0)
    idx = jnp.concatenate([halves[0][1], halves[1][1]], axis=0)
    return vals, idx
